# baseline (speedup 1.0000x reference)
import jax
import jax.numpy as jnp
from jax import lax
from jax.experimental import pallas as pl
from jax.experimental.pallas import tpu as pltpu
from jax.experimental.pallas import tpu_sc as plsc

NC = 2
NS = 16
L = 16
NW = NC * NS


def _make_body(rows_per_w, d, b):
    nblk = rows_per_w // b
    nch = d // L
    blk_elems = b * d

    def compute(perm_v, in_v, out_v):
        def chunk(i, carry):
            idx = perm_v[pl.ds(i * L, L)]
            for r in range(b):
                out_v[pl.ds(r * d + i * L, L)] = plsc.load_gather(
                    in_v, [idx + (r * d)])
            return carry

        lax.fori_loop(0, nch, chunk, 0, unroll=False)

    def body(x_hbm, perm_hbm, out_hbm, perm_v,
             in0, in1, out0, out1, sin0, sin1, sout0, sout1):
        c = lax.axis_index("c")
        s = lax.axis_index("s")
        wid = s * NC + c
        base = wid * rows_per_w * d
        pltpu.sync_copy(perm_hbm, perm_v)
        ins, outs = (in0, in1), (out0, out1)
        sins, souts = (sin0, sin1), (sout0, sout1)

        def in_copy(blk, j):
            return pltpu.make_async_copy(
                x_hbm.at[pl.ds(base + blk * blk_elems, blk_elems)],
                ins[j], sins[j])

        def out_copy(blk, j):
            return pltpu.make_async_copy(
                outs[j],
                out_hbm.at[pl.ds(base + blk * blk_elems, blk_elems)],
                souts[j])

        in_copy(0, 0).start()
        in_copy(1, 1).start()

        def pair(t, carry):
            for j in range(2):
                blk = t * 2 + j
                in_copy(blk, j).wait()

                @pl.when(blk >= 2)
                def _wait_out():
                    out_copy(blk - 2, j).wait()

                compute(perm_v, ins[j], outs[j])
                out_copy(blk, j).start()

                @pl.when(blk + 2 < nblk)
                def _prefetch():
                    in_copy(blk + 2, j).start()
            return carry

        lax.fori_loop(0, nblk // 2, pair, 0, unroll=False)
        out_copy(nblk - 2, 0).wait()
        out_copy(nblk - 1, 1).wait()

    return body


def _permute(x2, perm, rows, d, b):
    rows_per_w = rows // NW
    body = _make_body(rows_per_w, d, b)
    return pl.kernel(
        body,
        out_type=jax.ShapeDtypeStruct((rows * d,), jnp.float32),
        mesh=plsc.VectorSubcoreMesh(
            core_axis_name="c", subcore_axis_name="s",
            num_cores=NC, num_subcores=NS),
        scratch_types=[
            pltpu.VMEM((d,), jnp.int32),
            pltpu.VMEM((b * d,), jnp.float32),
            pltpu.VMEM((b * d,), jnp.float32),
            pltpu.VMEM((b * d,), jnp.float32),
            pltpu.VMEM((b * d,), jnp.float32),
            pltpu.SemaphoreType.DMA,
            pltpu.SemaphoreType.DMA,
            pltpu.SemaphoreType.DMA,
            pltpu.SemaphoreType.DMA,
        ],
        compiler_params=pltpu.CompilerParams(needs_layout_passes=False),
    )(x2, perm)


def kernel(x, permutation):
    lead = x.shape[:-1]
    d = x.shape[-1]
    rows = 1
    for n in lead:
        rows *= n
    x2 = x.reshape(rows * d)
    perm = permutation.astype(jnp.int32)
    out = _permute(x2, perm, rows, d, b=8)
    return out.reshape(x.shape)


# speedup vs baseline: 4.8708x; 4.8708x over previous
import jax
import jax.numpy as jnp
from jax import lax
from jax.experimental import pallas as pl
from jax.experimental.pallas import tpu as pltpu
from jax.experimental.pallas import tpu_sc as plsc

NC = 2
NS = 16
L = 16
NW = NC * NS


def _make_body(rows_per_w, d, b):
    nblk = rows_per_w // b
    nch = d // L
    blk_elems = b * d

    def compute(perm_v, in_v, out_v):
        @plsc.parallel_loop(0, nch, unroll=4)
        def _chunk(i):
            idx = perm_v[pl.ds(i * L, L)]
            for r in range(b):
                row = jnp.full((L,), r, dtype=jnp.int32)
                out_v[r, pl.ds(i * L, L)] = plsc.load_gather(
                    in_v, [row, idx])

    def body(x_hbm, perm_hbm, out_hbm, perm_v,
             in0, in1, out0, out1, sin0, sin1, sout0, sout1):
        c = lax.axis_index("c")
        s = lax.axis_index("s")
        wid = s * NC + c
        base = wid * rows_per_w
        pltpu.sync_copy(perm_hbm, perm_v)
        ins, outs = (in0, in1), (out0, out1)
        sins, souts = (sin0, sin1), (sout0, sout1)

        def in_copy(blk, j):
            return pltpu.make_async_copy(
                x_hbm.at[pl.ds(base + blk * b, b), :],
                ins[j], sins[j])

        def out_copy(blk, j):
            return pltpu.make_async_copy(
                outs[j],
                out_hbm.at[pl.ds(base + blk * b, b), :],
                souts[j])

        in_copy(0, 0).start()
        in_copy(1, 1).start()

        def pair(t, carry):
            for j in range(2):
                blk = t * 2 + j
                in_copy(blk, j).wait()

                @pl.when(blk >= 2)
                def _wait_out():
                    out_copy(blk - 2, j).wait()

                compute(perm_v, ins[j], outs[j])
                out_copy(blk, j).start()

                @pl.when(blk + 2 < nblk)
                def _prefetch():
                    in_copy(blk + 2, j).start()
            return carry

        lax.fori_loop(0, nblk // 2, pair, 0, unroll=False)
        out_copy(nblk - 2, 0).wait()
        out_copy(nblk - 1, 1).wait()

    return body


def _permute(x2, perm, rows, d, b):
    rows_per_w = rows // NW
    body = _make_body(rows_per_w, d, b)
    return pl.kernel(
        body,
        out_type=jax.ShapeDtypeStruct((rows, d), jnp.float32),
        mesh=plsc.VectorSubcoreMesh(
            core_axis_name="c", subcore_axis_name="s",
            num_cores=NC, num_subcores=NS),
        scratch_types=[
            pltpu.VMEM((d,), jnp.int32),
            pltpu.VMEM((b, d), jnp.float32),
            pltpu.VMEM((b, d), jnp.float32),
            pltpu.VMEM((b, d), jnp.float32),
            pltpu.VMEM((b, d), jnp.float32),
            pltpu.SemaphoreType.DMA,
            pltpu.SemaphoreType.DMA,
            pltpu.SemaphoreType.DMA,
            pltpu.SemaphoreType.DMA,
        ],
        compiler_params=pltpu.CompilerParams(needs_layout_passes=False),
    )(x2, perm)


def kernel(x, permutation):
    lead = x.shape[:-1]
    d = x.shape[-1]
    rows = 1
    for n in lead:
        rows *= n
    x2 = x.reshape(rows, d)
    perm = permutation.astype(jnp.int32)
    out = _permute(x2, perm, rows, d, b=8)
    return out.reshape(x.shape)
